# Initial kernel scaffold; baseline (speedup 1.0000x reference)
import jax
import jax.numpy as jnp
from jax.experimental import pallas as pl


def kernel(x, edge_index, W):
    raise NotImplementedError("write your pallas kernel here")


# speedup vs baseline: 1.4113x; 1.4113x over previous
import functools

import jax
import jax.numpy as jnp
from jax import lax
from jax.experimental import pallas as pl
from jax.experimental.pallas import tpu as pltpu
from jax.experimental.pallas import tpu_sc as plsc

N = 10000
K = 32
D_IN = 128
D_OUT = 128

NUM_CORES = 2
NUM_SUBCORES = 16
NUM_TILES = NUM_CORES * NUM_SUBCORES

W_NODES = 4
IDX_PER_STEP = W_NODES * K
NODES_PER_TILE = -(-N // NUM_TILES)
NODES_PER_TILE += (-NODES_PER_TILE) % 8
NP = NUM_TILES * NODES_PER_TILE
NSTEPS = NODES_PER_TILE // W_NODES

BA = 640
LANES = 16
NCHUNK = D_OUT // LANES


def _stage_body(x_ref, e_ref, w_ref, xt_ref, ce_ref):
    e = e_ref[...]
    deg = jnp.sum((e >= 0).astype(jnp.float32), axis=1, keepdims=True) + 1.0
    s = lax.rsqrt(deg)
    acc = lax.dot_general(
        x_ref[...], w_ref[...], (((1,), (1,)), ((), ())),
        preferred_element_type=jnp.float32,
        precision=lax.Precision.HIGHEST,
    )
    xt_ref[...] = acc * s
    ce_ref[...] = jnp.where(e < 0, N, e)


def _final_body(g_ref, xt_ref, ce_ref, o_ref):
    deg = jnp.sum((ce_ref[...] < N).astype(jnp.float32), axis=1,
                  keepdims=True) + 1.0
    o_ref[...] = (g_ref[...] + xt_ref[...]) * lax.rsqrt(deg)


_SC_MESH = plsc.VectorSubcoreMesh(core_axis_name="c", subcore_axis_name="s",
                                  num_cores=NUM_CORES,
                                  num_subcores=NUM_SUBCORES)


@functools.partial(
    pl.kernel,
    out_type=jax.ShapeDtypeStruct((NP, D_OUT), jnp.float32),
    mesh=_SC_MESH,
    scratch_types=[
        pltpu.VMEM((IDX_PER_STEP,), jnp.int32),
        pltpu.VMEM((IDX_PER_STEP,), jnp.int32),
        pltpu.VMEM((IDX_PER_STEP, D_OUT), jnp.float32),
        pltpu.VMEM((IDX_PER_STEP, D_OUT), jnp.float32),
        pltpu.VMEM((NODES_PER_TILE, D_OUT), jnp.float32),
        pltpu.SemaphoreType.DMA,
        pltpu.SemaphoreType.DMA,
    ],
)
def _gather_sum(xt_hbm, ce_hbm, out_hbm, idx0, idx1, rows0, rows1,
                outbuf, sem0, sem1):
    wid = lax.axis_index("s") * NUM_CORES + lax.axis_index("c")
    node_base = wid * NODES_PER_TILE
    idx_base = node_base * K

    def load_idx(j, ib):
        pltpu.sync_copy(ce_hbm.at[pl.ds(idx_base + j * IDX_PER_STEP,
                                        IDX_PER_STEP)], ib)

    def start_gather(ib, rb, sem):
        pltpu.async_copy(xt_hbm.at[ib], rb, sem)

    def wait_gather(ib, rb, sem):
        pltpu.make_async_copy(xt_hbm.at[ib], rb, sem).wait()

    def reduce_step(j, rb):
        @pl.loop(0, W_NODES)
        def _(w):
            row0 = w * K
            init = tuple(rb[row0, pl.ds(c * LANES, LANES)]
                         for c in range(NCHUNK))

            def body(k, accs):
                return tuple(accs[c] + rb[row0 + k, pl.ds(c * LANES, LANES)]
                             for c in range(NCHUNK))

            accs = lax.fori_loop(1, K, body, init)
            node = j * W_NODES + w
            for c in range(NCHUNK):
                outbuf[node, pl.ds(c * LANES, LANES)] = accs[c]

    load_idx(0, idx0)
    start_gather(idx0, rows0, sem0)
    load_idx(1, idx1)

    @pl.loop(0, NSTEPS, step=2)
    def _(j):
        @pl.when(j + 1 < NSTEPS)
        def _():
            start_gather(idx1, rows1, sem1)

        wait_gather(idx0, rows0, sem0)

        @pl.when(j + 2 < NSTEPS)
        def _():
            load_idx(j + 2, idx0)

        reduce_step(j, rows0)

        @pl.when(j + 1 < NSTEPS)
        def _():
            wait_gather(idx1, rows1, sem1)

            @pl.when(j + 3 < NSTEPS)
            def _():
                load_idx(j + 3, idx1)

            @pl.when(j + 2 < NSTEPS)
            def _():
                start_gather(idx0, rows0, sem0)

            reduce_step(j + 1, rows1)

    pltpu.sync_copy(outbuf, out_hbm.at[pl.ds(node_base, NODES_PER_TILE)])


def kernel(x, edge_index, W):
    e32 = edge_index.astype(jnp.int32)
    x_p = jnp.pad(x, ((0, NP - N), (0, 0)))
    e_p = jnp.pad(e32, ((0, NP - N), (0, 0)))

    grid = (NP // BA,)
    xt, ce = pl.pallas_call(
        _stage_body,
        grid=grid,
        in_specs=[
            pl.BlockSpec((BA, D_IN), lambda i: (i, 0)),
            pl.BlockSpec((BA, K), lambda i: (i, 0)),
            pl.BlockSpec((D_OUT, D_IN), lambda i: (0, 0)),
        ],
        out_specs=[
            pl.BlockSpec((BA, D_OUT), lambda i: (i, 0)),
            pl.BlockSpec((BA, K), lambda i: (i, 0)),
        ],
        out_shape=[
            jax.ShapeDtypeStruct((NP, D_OUT), jnp.float32),
            jax.ShapeDtypeStruct((NP, K), jnp.int32),
        ],
    )(x_p, e_p, W)

    gsum = _gather_sum(xt, ce.reshape(-1))

    out = pl.pallas_call(
        _final_body,
        grid=grid,
        in_specs=[
            pl.BlockSpec((BA, D_OUT), lambda i: (i, 0)),
            pl.BlockSpec((BA, D_OUT), lambda i: (i, 0)),
            pl.BlockSpec((BA, K), lambda i: (i, 0)),
        ],
        out_specs=pl.BlockSpec((BA, D_OUT), lambda i: (i, 0)),
        out_shape=jax.ShapeDtypeStruct((NP, D_OUT), jnp.float32),
    )(gsum, xt, ce)

    return out[:N]


# speedup vs baseline: 1.4664x; 1.0390x over previous
import functools

import jax
import jax.numpy as jnp
from jax import lax
from jax.experimental import pallas as pl
from jax.experimental.pallas import tpu as pltpu
from jax.experimental.pallas import tpu_sc as plsc

N = 10000
K = 32
D_IN = 128
D_OUT = 128

NUM_CORES = 2
NUM_SUBCORES = 16
NUM_TILES = NUM_CORES * NUM_SUBCORES

W_NODES = 4
IDX_PER_STEP = W_NODES * K
NODES_PER_TILE = -(-N // NUM_TILES)
NODES_PER_TILE += (-NODES_PER_TILE) % 8
NP = NUM_TILES * NODES_PER_TILE
NSTEPS = NODES_PER_TILE // W_NODES

BA = 640
LANES = 16
NCHUNK = D_OUT // LANES


def _stage_body(x_ref, e_ref, w_ref, xt_ref, ce_ref):
    e = e_ref[...]
    deg = jnp.sum((e >= 0).astype(jnp.float32), axis=1, keepdims=True) + 1.0
    s = lax.rsqrt(deg)
    acc = lax.dot_general(
        x_ref[...], w_ref[...], (((1,), (1,)), ((), ())),
        preferred_element_type=jnp.float32,
        precision=lax.Precision.HIGHEST,
    )
    xt_ref[...] = acc * s
    ce_ref[...] = jnp.where(e < 0, N, e)


def _final_body(g_ref, xt_ref, ce_ref, o_ref):
    deg = jnp.sum((ce_ref[...] < N).astype(jnp.float32), axis=1,
                  keepdims=True) + 1.0
    o_ref[...] = (g_ref[...] + xt_ref[...]) * lax.rsqrt(deg)


_SC_MESH = plsc.VectorSubcoreMesh(core_axis_name="c", subcore_axis_name="s",
                                  num_cores=NUM_CORES,
                                  num_subcores=NUM_SUBCORES)


NBUF = 4


@functools.partial(
    pl.kernel,
    out_type=jax.ShapeDtypeStruct((NP, D_OUT), jnp.float32),
    mesh=_SC_MESH,
    scratch_types=[
        pltpu.VMEM((NODES_PER_TILE * K,), jnp.int32),
        pltpu.VMEM((IDX_PER_STEP, D_OUT), jnp.float32),
        pltpu.VMEM((IDX_PER_STEP, D_OUT), jnp.float32),
        pltpu.VMEM((IDX_PER_STEP, D_OUT), jnp.float32),
        pltpu.VMEM((IDX_PER_STEP, D_OUT), jnp.float32),
        pltpu.VMEM((NODES_PER_TILE, D_OUT), jnp.float32),
        pltpu.SemaphoreType.DMA,
        pltpu.SemaphoreType.DMA,
        pltpu.SemaphoreType.DMA,
        pltpu.SemaphoreType.DMA,
    ],
)
def _gather_sum(xt_hbm, ce_hbm, out_hbm, idx_all, rows0, rows1, rows2,
                rows3, outbuf, sem0, sem1, sem2, sem3):
    rows = (rows0, rows1, rows2, rows3)
    sems = (sem0, sem1, sem2, sem3)
    wid = lax.axis_index("s") * NUM_CORES + lax.axis_index("c")
    node_base = wid * NODES_PER_TILE

    pltpu.sync_copy(ce_hbm.at[pl.ds(node_base * K, NODES_PER_TILE * K)],
                    idx_all)

    def start_gather(j, b):
        pltpu.async_copy(
            xt_hbm.at[idx_all.at[pl.ds(j * IDX_PER_STEP, IDX_PER_STEP)]],
            rows[b], sems[b])

    def wait_gather(b):
        pltpu.make_async_copy(
            xt_hbm.at[idx_all.at[pl.ds(0, IDX_PER_STEP)]],
            rows[b], sems[b]).wait()

    def reduce_step(j, b):
        rb = rows[b]

        @pl.loop(0, W_NODES)
        def _(w):
            row0 = w * K
            accs = [rb[row0, pl.ds(c * LANES, LANES)] for c in range(NCHUNK)]
            for k in range(1, K):
                for c in range(NCHUNK):
                    accs[c] = accs[c] + rb[row0 + k, pl.ds(c * LANES, LANES)]
            node = j * W_NODES + w
            for c in range(NCHUNK):
                outbuf[node, pl.ds(c * LANES, LANES)] = accs[c]

    for b in range(NBUF):
        start_gather(b, b)

    @pl.loop(0, NSTEPS, step=NBUF)
    def _(j):
        for b in range(NBUF):
            wait_gather(b)
            reduce_step(j + b, b)

            @pl.when(j + b + NBUF < NSTEPS)
            def _():
                start_gather(j + b + NBUF, b)

    pltpu.sync_copy(outbuf, out_hbm.at[pl.ds(node_base, NODES_PER_TILE)])


def kernel(x, edge_index, W):
    e32 = edge_index.astype(jnp.int32)
    x_p = jnp.pad(x, ((0, NP - N), (0, 0)))
    e_p = jnp.pad(e32, ((0, NP - N), (0, 0)))

    grid = (NP // BA,)
    xt, ce = pl.pallas_call(
        _stage_body,
        grid=grid,
        in_specs=[
            pl.BlockSpec((BA, D_IN), lambda i: (i, 0)),
            pl.BlockSpec((BA, K), lambda i: (i, 0)),
            pl.BlockSpec((D_OUT, D_IN), lambda i: (0, 0)),
        ],
        out_specs=[
            pl.BlockSpec((BA, D_OUT), lambda i: (i, 0)),
            pl.BlockSpec((BA, K), lambda i: (i, 0)),
        ],
        out_shape=[
            jax.ShapeDtypeStruct((NP, D_OUT), jnp.float32),
            jax.ShapeDtypeStruct((NP, K), jnp.int32),
        ],
    )(x_p, e_p, W)

    gsum = _gather_sum(xt, ce.reshape(-1))

    out = pl.pallas_call(
        _final_body,
        grid=grid,
        in_specs=[
            pl.BlockSpec((BA, D_OUT), lambda i: (i, 0)),
            pl.BlockSpec((BA, D_OUT), lambda i: (i, 0)),
            pl.BlockSpec((BA, K), lambda i: (i, 0)),
        ],
        out_specs=pl.BlockSpec((BA, D_OUT), lambda i: (i, 0)),
        out_shape=jax.ShapeDtypeStruct((NP, D_OUT), jnp.float32),
    )(gsum, xt, ce)

    return out[:N]


# speedup vs baseline: 4.1629x; 2.8389x over previous
import functools

import jax
import jax.numpy as jnp
from jax import lax
from jax.experimental import pallas as pl
from jax.experimental.pallas import tpu as pltpu
from jax.experimental.pallas import tpu_sc as plsc

N = 10000
K = 32
D_IN = 128
D_OUT = 128

NUM_CORES = 2
NUM_SUBCORES = 16
NUM_TILES = NUM_CORES * NUM_SUBCORES

W_NODES = 2
IDX_PER_STEP = W_NODES * K
NODES_PER_TILE = -(-N // NUM_TILES)
NODES_PER_TILE += (-NODES_PER_TILE) % 8
NP = NUM_TILES * NODES_PER_TILE
NSTEPS = NODES_PER_TILE // W_NODES

BA = 640
LANES = 16
NCHUNK = D_OUT // LANES


def _stage_body(x_ref, e_ref, w_ref, xt_ref, ce_ref):
    e = e_ref[...]
    deg = jnp.sum((e >= 0).astype(jnp.float32), axis=1, keepdims=True) + 1.0
    s = lax.rsqrt(deg)
    acc = lax.dot_general(
        x_ref[...], w_ref[...], (((1,), (1,)), ((), ())),
        preferred_element_type=jnp.float32,
        precision=lax.Precision.HIGHEST,
    )
    xt_ref[...] = acc * s
    ce_ref[...] = jnp.where(e < 0, N, e)


def _final_body(g_ref, xt_ref, ce_ref, o_ref):
    deg = jnp.sum((ce_ref[...] < N).astype(jnp.float32), axis=1,
                  keepdims=True) + 1.0
    o_ref[...] = (g_ref[...] + xt_ref[...]) * lax.rsqrt(deg)


_SC_MESH = plsc.VectorSubcoreMesh(core_axis_name="c", subcore_axis_name="s",
                                  num_cores=NUM_CORES,
                                  num_subcores=NUM_SUBCORES)


STEPS_PER_BLK = 4
OUT_BLK = STEPS_PER_BLK * W_NODES


@functools.partial(
    pl.kernel,
    out_type=jax.ShapeDtypeStruct((NP, D_OUT), jnp.float32),
    mesh=_SC_MESH,
    scratch_types=[
        pltpu.VMEM((NODES_PER_TILE * K,), jnp.int32),
        pltpu.VMEM((IDX_PER_STEP, D_OUT), jnp.float32),
        pltpu.VMEM((IDX_PER_STEP, D_OUT), jnp.float32),
        pltpu.VMEM((OUT_BLK, D_OUT), jnp.float32),
        pltpu.VMEM((OUT_BLK, D_OUT), jnp.float32),
        pltpu.VMEM_SHARED((NP, D_OUT), jnp.float32),
        pltpu.SemaphoreType.DMA,
        pltpu.SemaphoreType.DMA,
        pltpu.SemaphoreType.DMA,
    ],
)
def _gather_sum(xt_hbm, ce_hbm, out_hbm, idx_all, rows0, rows1, outb0,
                outb1, table_sp, sem0, sem1, osem):
    rows = (rows0, rows1)
    sems = (sem0, sem1)
    outs = (outb0, outb1)
    sid = lax.axis_index("s")
    wid = sid * NUM_CORES + lax.axis_index("c")
    node_base = wid * NODES_PER_TILE

    pltpu.sync_copy(ce_hbm.at[pl.ds(node_base * K, NODES_PER_TILE * K)],
                    idx_all)
    stage_rows = NP // NUM_SUBCORES
    pltpu.sync_copy(xt_hbm.at[pl.ds(sid * stage_rows, stage_rows)],
                    table_sp.at[pl.ds(sid * stage_rows, stage_rows)])
    plsc.subcore_barrier()

    def start_gather(j, b):
        pltpu.async_copy(
            table_sp.at[idx_all.at[pl.ds(j * IDX_PER_STEP, IDX_PER_STEP)]],
            rows[b], sems[b])

    def wait_gather(b):
        pltpu.make_async_copy(
            table_sp.at[idx_all.at[pl.ds(0, IDX_PER_STEP)]],
            rows[b], sems[b]).wait()

    def reduce_step(b, ob, t):
        rb = rows[b]

        @pl.loop(0, W_NODES)
        def _(w):
            row0 = w * K
            accs = [rb[row0, pl.ds(c * LANES, LANES)] for c in range(NCHUNK)]
            for k in range(1, K):
                for c in range(NCHUNK):
                    accs[c] = accs[c] + rb[row0 + k, pl.ds(c * LANES, LANES)]
            for c in range(NCHUNK):
                ob[t * W_NODES + w, pl.ds(c * LANES, LANES)] = accs[c]

    def out_slice(j):
        return out_hbm.at[pl.ds(node_base + j * W_NODES, OUT_BLK)]

    def do_block(j, ob):
        @pl.when(j >= 2 * STEPS_PER_BLK)
        def _():
            pltpu.make_async_copy(ob, out_slice(j - 2 * STEPS_PER_BLK),
                                  osem).wait()

        for t in range(STEPS_PER_BLK):
            b = t & 1
            wait_gather(b)
            reduce_step(b, ob, t)

            @pl.when(j + t + 2 < NSTEPS)
            def _():
                start_gather(j + t + 2, b)

        pltpu.async_copy(ob, out_slice(j), osem)

    start_gather(0, 0)
    start_gather(1, 1)

    @pl.loop(0, NSTEPS, step=2 * STEPS_PER_BLK)
    def _(j):
        do_block(j, outs[0])
        do_block(j + STEPS_PER_BLK, outs[1])

    pltpu.make_async_copy(outs[0], out_slice(NSTEPS - 2 * STEPS_PER_BLK),
                          osem).wait()
    pltpu.make_async_copy(outs[1], out_slice(NSTEPS - STEPS_PER_BLK),
                          osem).wait()


def kernel(x, edge_index, W):
    e32 = edge_index.astype(jnp.int32)
    x_p = jnp.pad(x, ((0, NP - N), (0, 0)))
    e_p = jnp.pad(e32, ((0, NP - N), (0, 0)))

    grid = (NP // BA,)
    xt, ce = pl.pallas_call(
        _stage_body,
        grid=grid,
        in_specs=[
            pl.BlockSpec((BA, D_IN), lambda i: (i, 0)),
            pl.BlockSpec((BA, K), lambda i: (i, 0)),
            pl.BlockSpec((D_OUT, D_IN), lambda i: (0, 0)),
        ],
        out_specs=[
            pl.BlockSpec((BA, D_OUT), lambda i: (i, 0)),
            pl.BlockSpec((BA, K), lambda i: (i, 0)),
        ],
        out_shape=[
            jax.ShapeDtypeStruct((NP, D_OUT), jnp.float32),
            jax.ShapeDtypeStruct((NP, K), jnp.int32),
        ],
    )(x_p, e_p, W)

    gsum = _gather_sum(xt, ce.reshape(-1))

    out = pl.pallas_call(
        _final_body,
        grid=grid,
        in_specs=[
            pl.BlockSpec((BA, D_OUT), lambda i: (i, 0)),
            pl.BlockSpec((BA, D_OUT), lambda i: (i, 0)),
            pl.BlockSpec((BA, K), lambda i: (i, 0)),
        ],
        out_specs=pl.BlockSpec((BA, D_OUT), lambda i: (i, 0)),
        out_shape=jax.ShapeDtypeStruct((NP, D_OUT), jnp.float32),
    )(gsum, xt, ce)

    return out[:N]


# speedup vs baseline: 5.5863x; 1.3419x over previous
import dataclasses
import functools

import jax
import jax.numpy as jnp
from jax import lax
from jax.experimental import pallas as pl
from jax.experimental.pallas import tpu as pltpu
from jax.experimental.pallas import tpu_sc as plsc

N = 10000
K = 32
D_IN = 128
D_OUT = 128

NUM_CORES = 2
NUM_SUBCORES = 16
NUM_TILES = NUM_CORES * NUM_SUBCORES

W_NODES = 2
IDX_PER_STEP = W_NODES * K
NODES_PER_TILE = -(-N // NUM_TILES)
NODES_PER_TILE += (-NODES_PER_TILE) % 8
NP = NUM_TILES * NODES_PER_TILE
NSTEPS = NODES_PER_TILE // W_NODES

BA = 640
LANES = 16
NCHUNK = D_OUT // LANES


def _stage_body(x_ref, e_ref, w_ref, xt_ref, ce_ref, s_ref):
    e = e_ref[...]
    deg = jnp.sum((e >= 0).astype(jnp.float32), axis=1, keepdims=True) + 1.0
    s = lax.rsqrt(deg)
    acc = lax.dot_general(
        x_ref[...], w_ref[...], (((1,), (1,)), ((), ())),
        preferred_element_type=jnp.float32,
        precision=lax.Precision.HIGHEST,
    )
    xt_ref[...] = acc * s
    ce_ref[...] = jnp.where(e < 0, N, e)
    s_ref[...] = s


_SC_MESH = plsc.VectorSubcoreMesh(core_axis_name="c", subcore_axis_name="s",
                                  num_cores=NUM_CORES,
                                  num_subcores=NUM_SUBCORES)

_SC_PARAMS = pltpu.CompilerParams()
if "needs_layout_passes" in pltpu.CompilerParams.__dataclass_fields__:
    _SC_PARAMS = dataclasses.replace(_SC_PARAMS, needs_layout_passes=False)


STEPS_PER_BLK = 4
OUT_BLK = STEPS_PER_BLK * W_NODES


@functools.partial(
    pl.kernel,
    out_type=jax.ShapeDtypeStruct((NP, D_OUT), jnp.float32),
    mesh=_SC_MESH,
    compiler_params=_SC_PARAMS,
    scratch_types=[
        pltpu.VMEM((NODES_PER_TILE * K,), jnp.int32),
        pltpu.VMEM((IDX_PER_STEP, D_OUT), jnp.float32),
        pltpu.VMEM((IDX_PER_STEP, D_OUT), jnp.float32),
        pltpu.VMEM((OUT_BLK, D_OUT), jnp.float32),
        pltpu.VMEM((OUT_BLK, D_OUT), jnp.float32),
        pltpu.VMEM((OUT_BLK, D_OUT), jnp.float32),
        pltpu.VMEM((NODES_PER_TILE,), jnp.float32),
        pltpu.VMEM_SHARED((NP, D_OUT), jnp.float32),
        pltpu.SemaphoreType.DMA,
        pltpu.SemaphoreType.DMA,
        pltpu.SemaphoreType.DMA,
    ],
)
def _gather_sum(xt_hbm, ce_hbm, s_hbm, out_hbm, idx_all, rows0, rows1,
                outb0, outb1, ownbuf, s_vmem, table_sp, sem0, sem1, osem):
    rows = (rows0, rows1)
    sems = (sem0, sem1)
    outs = (outb0, outb1)
    sid = lax.axis_index("s")
    wid = sid * NUM_CORES + lax.axis_index("c")
    node_base = wid * NODES_PER_TILE

    pltpu.sync_copy(ce_hbm.at[pl.ds(node_base * K, NODES_PER_TILE * K)],
                    idx_all)
    pltpu.sync_copy(s_hbm.at[pl.ds(node_base, NODES_PER_TILE)], s_vmem)
    stage_rows = NP // NUM_SUBCORES
    pltpu.sync_copy(xt_hbm.at[pl.ds(sid * stage_rows, stage_rows)],
                    table_sp.at[pl.ds(sid * stage_rows, stage_rows)])
    plsc.subcore_barrier()

    def start_gather(j, b):
        pltpu.async_copy(
            table_sp.at[idx_all.at[pl.ds(j * IDX_PER_STEP, IDX_PER_STEP)]],
            rows[b], sems[b])

    def wait_gather(b):
        pltpu.make_async_copy(
            table_sp.at[idx_all.at[pl.ds(0, IDX_PER_STEP)]],
            rows[b], sems[b]).wait()

    def reduce_step(j, b, ob, t):
        rb = rows[b]

        @pl.loop(0, W_NODES)
        def _(w):
            row0 = w * K
            blk_row = t * W_NODES + w
            node_local = (j + t) * W_NODES + w
            sval = plsc.load_gather(
                s_vmem, [jnp.full((LANES,), node_local, jnp.int32)])
            accs = [ownbuf[blk_row, pl.ds(c * LANES, LANES)]
                    for c in range(NCHUNK)]

            def body(kk, carry):
                accs_in = list(carry)
                for r in range(K // 2):
                    for c in range(NCHUNK):
                        accs_in[c] = accs_in[c] + rb[
                            row0 + kk * (K // 2) + r, pl.ds(c * LANES, LANES)]
                return tuple(accs_in)

            accs = lax.fori_loop(0, 2, body, tuple(accs))
            for c in range(NCHUNK):
                ob[blk_row, pl.ds(c * LANES, LANES)] = accs[c] * sval

    def out_slice(j):
        return out_hbm.at[pl.ds(node_base + j * W_NODES, OUT_BLK)]

    def do_block(j, ob):
        @pl.when(j >= 2 * STEPS_PER_BLK)
        def _():
            pltpu.make_async_copy(ob, out_slice(j - 2 * STEPS_PER_BLK),
                                  osem).wait()

        pltpu.sync_copy(
            table_sp.at[pl.ds(node_base + j * W_NODES, OUT_BLK)], ownbuf)

        for t in range(STEPS_PER_BLK):
            b = t & 1
            wait_gather(b)
            reduce_step(j, b, ob, t)

            @pl.when(j + t + 2 < NSTEPS)
            def _():
                start_gather(j + t + 2, b)

        pltpu.async_copy(ob, out_slice(j), osem)

    start_gather(0, 0)
    start_gather(1, 1)

    @pl.loop(0, NSTEPS, step=2 * STEPS_PER_BLK)
    def _(j):
        do_block(j, outs[0])
        do_block(j + STEPS_PER_BLK, outs[1])

    pltpu.make_async_copy(outs[0], out_slice(NSTEPS - 2 * STEPS_PER_BLK),
                          osem).wait()
    pltpu.make_async_copy(outs[1], out_slice(NSTEPS - STEPS_PER_BLK),
                          osem).wait()


def kernel(x, edge_index, W):
    e32 = edge_index.astype(jnp.int32)
    x_p = jnp.pad(x, ((0, NP - N), (0, 0)))
    e_p = jnp.pad(e32, ((0, NP - N), (0, 0)))

    grid = (NP // BA,)
    xt, ce, s = pl.pallas_call(
        _stage_body,
        grid=grid,
        in_specs=[
            pl.BlockSpec((BA, D_IN), lambda i: (i, 0)),
            pl.BlockSpec((BA, K), lambda i: (i, 0)),
            pl.BlockSpec((D_OUT, D_IN), lambda i: (0, 0)),
        ],
        out_specs=[
            pl.BlockSpec((BA, D_OUT), lambda i: (i, 0)),
            pl.BlockSpec((BA, K), lambda i: (i, 0)),
            pl.BlockSpec((BA, 1), lambda i: (i, 0)),
        ],
        out_shape=[
            jax.ShapeDtypeStruct((NP, D_OUT), jnp.float32),
            jax.ShapeDtypeStruct((NP, K), jnp.int32),
            jax.ShapeDtypeStruct((NP, 1), jnp.float32),
        ],
    )(x_p, e_p, W)

    out = _gather_sum(xt, ce.reshape(-1), s.reshape(-1))

    return out[:N]
